# Initial kernel scaffold; baseline (speedup 1.0000x reference)
import jax
import jax.numpy as jnp
from jax.experimental import pallas as pl


def kernel(x, edge_index, deg, W_self, b_self, W_neigh, b_neigh):
    raise NotImplementedError("write your pallas kernel here")


# speedup vs baseline: 2.8955x; 2.8955x over previous
import dataclasses
import functools

import jax
import jax.numpy as jnp
from jax import lax
from jax.experimental import pallas as pl
from jax.experimental.pallas import tpu as pltpu
from jax.experimental.pallas import tpu_sc as plsc

NC = 1
NS = 16
NW = NC * NS
CH = 80


def _sc_segment_sum(src, dst, x2d, zx, n_pad):
    e = src.shape[0]
    d = x2d.shape[1]
    epw = e // NW
    mesh = plsc.VectorSubcoreMesh(core_axis_name="c", subcore_axis_name="s", num_cores=NC)
    rows_per_sub = n_pad // NS

    cp = pltpu.CompilerParams()
    if "needs_layout_passes" in pltpu.CompilerParams.__dataclass_fields__:
        cp = dataclasses.replace(cp, needs_layout_passes=False)

    @functools.partial(
        pl.kernel,
        compiler_params=cp,
        out_type=[
            jax.ShapeDtypeStruct((NC, n_pad, d), jnp.float32),
            jax.ShapeDtypeStruct((NC * n_pad,), jnp.float32),
        ],
        mesh=mesh,
        scratch_types=[
            pltpu.VMEM((CH,), jnp.int32),
            pltpu.VMEM((CH,), jnp.int32),
            pltpu.VMEM((CH, d), jnp.float32),
            pltpu.VMEM((n_pad,), jnp.float32),
            pltpu.VMEM((rows_per_sub,), jnp.float32),
            pltpu.VMEM((rows_per_sub,), jnp.float32),
            pltpu.VMEM_SHARED((n_pad, d), jnp.float32),
            pltpu.VMEM_SHARED((NS, n_pad), jnp.float32),
            pltpu.SemaphoreType.DMA,
        ],
    )
    def seg_sum(src_hbm, dst_hbm, x_hbm, zx_hbm, accx_hbm, cnt_hbm,
                sidx_v, didx_v, rows_v, cnt_tile, red_v, rin_v,
                accx_sh, cstage_sh, sem):
        c = lax.axis_index("c")
        s = lax.axis_index("s")
        rbase = s * rows_per_sub

        z16 = jnp.zeros((16,), jnp.float32)
        one16 = jnp.full((16,), 1.0, jnp.float32)

        @pl.loop(0, n_pad, step=16)
        def _(r):
            cnt_tile.at[pl.ds(r, 16)][...] = z16

        pltpu.sync_copy(zx_hbm, accx_sh.at[pl.ds(rbase, rows_per_sub)])
        plsc.subcore_barrier()

        wbase = (c * NS + s) * epw

        @pl.loop(0, epw, step=CH)
        def _(i):
            base = wbase + i
            pltpu.sync_copy(src_hbm.at[pl.ds(base, CH)], sidx_v)
            pltpu.sync_copy(dst_hbm.at[pl.ds(base, CH)], didx_v)
            pltpu.async_copy(x_hbm.at[sidx_v], rows_v, sem).wait()
            pltpu.sync_copy(rows_v, accx_sh.at[didx_v], add=True)
            @pl.loop(0, CH, step=16)
            def _(j):
                idx16 = didx_v.at[pl.ds(j, 16)][...]
                plsc.addupdate_scatter(cnt_tile, [idx16], one16)

        plsc.subcore_barrier()

        pltpu.sync_copy(accx_sh.at[pl.ds(rbase, rows_per_sub)],
                        accx_hbm.at[c, pl.ds(rbase, rows_per_sub)])

        pltpu.sync_copy(cnt_tile, cstage_sh.at[s])
        plsc.subcore_barrier()

        @pl.loop(0, rows_per_sub, step=16)
        def _(r):
            red_v.at[pl.ds(r, 16)][...] = z16

        for k in range(NS):
            pltpu.sync_copy(cstage_sh.at[k, pl.ds(rbase, rows_per_sub)], rin_v)

            @pl.loop(0, rows_per_sub, step=16)
            def _(r):
                red_v.at[pl.ds(r, 16)][...] = (
                    red_v.at[pl.ds(r, 16)][...] + rin_v.at[pl.ds(r, 16)][...])

        pltpu.sync_copy(red_v,
                        cnt_hbm.at[pl.ds(c * n_pad + rbase, rows_per_sub)])

    return seg_sum(src, dst, x2d, zx)


def _tc_finish(x2d, a0, c0, degf, wsT, wnT, bs, bn, n, d, rows):
    grid = (n // rows,)

    def body(x_ref, a0_ref, c0_ref, deg_ref,
             wsT_ref, wnT_ref, bs_ref, bn_ref, o_ref):
        xv = x_ref[...]
        acc = a0_ref[...]
        cnt = c0_ref[...]
        denom = jnp.maximum(deg_ref[...], 1.0)
        inv = 1.0 / denom
        self_out = jnp.dot(xv, wsT_ref[...],
                           preferred_element_type=jnp.float32,
                           precision=lax.Precision.HIGHEST) + bs_ref[...]
        neigh = jnp.dot(acc, wnT_ref[...],
                        preferred_element_type=jnp.float32,
                        precision=lax.Precision.HIGHEST)
        out = self_out + (neigh + cnt * bn_ref[...]) * inv
        o_ref[...] = jnp.where(out >= 0, out, 0.1 * out)

    return pl.pallas_call(
        body,
        grid=grid,
        in_specs=[
            pl.BlockSpec((rows, d), lambda i: (i, 0)),
            pl.BlockSpec((rows, d), lambda i: (i, 0)),
            pl.BlockSpec((rows, 1), lambda i: (i, 0)),
            pl.BlockSpec((rows, 1), lambda i: (i, 0)),
            pl.BlockSpec((d, d), lambda i: (0, 0)),
            pl.BlockSpec((d, d), lambda i: (0, 0)),
            pl.BlockSpec((1, d), lambda i: (0, 0)),
            pl.BlockSpec((1, d), lambda i: (0, 0)),
        ],
        out_specs=pl.BlockSpec((rows, d), lambda i: (i, 0)),
        out_shape=jax.ShapeDtypeStruct((n, d), jnp.float32),
    )(x2d, a0, c0, degf, wsT, wnT, bs, bn)


def kernel(x, edge_index, deg, W_self, b_self, W_neigh, b_neigh):
    b, n, d = x.shape
    e = edge_index.shape[1]
    assert b == 1 and e % NW == 0 and (e // NW) % CH == 0

    src = edge_index[0].astype(jnp.int32)
    dst = edge_index[1].astype(jnp.int32)
    x2d = x.reshape(n, d).astype(jnp.float32)

    n_pad = ((n + NS * CH - 1) // (NS * CH)) * (NS * CH)
    rows_per_sub = n_pad // NS
    zx = jnp.zeros((rows_per_sub, d), jnp.float32)

    accx, cnt = _sc_segment_sum(src, dst, x2d, zx, n_pad)

    degf = deg.astype(jnp.float32).reshape(n, 1)
    out2d = _tc_finish(
        x2d, accx[0, :n],
        cnt[:n].reshape(n, 1), degf,
        W_self.T.astype(jnp.float32), W_neigh.T.astype(jnp.float32),
        b_self.reshape(1, d).astype(jnp.float32),
        b_neigh.reshape(1, d).astype(jnp.float32),
        n, d, rows=1000,
    )
    return out2d.reshape(b, n, d).astype(x.dtype)


# speedup vs baseline: 4.9487x; 1.7091x over previous
import dataclasses
import functools

import jax
import jax.numpy as jnp
from jax import lax
from jax.experimental import pallas as pl
from jax.experimental.pallas import tpu as pltpu
from jax.experimental.pallas import tpu_sc as plsc

NC = 2
NS = 16
NW = NC * NS
CH = 80


def _sc_segment_sum(src, dst, x2d, zx, n_pad):
    e = src.shape[0]
    d = x2d.shape[1]
    epw = e // NW
    mesh = plsc.VectorSubcoreMesh(core_axis_name="c", subcore_axis_name="s", num_cores=NC)
    rows_per_sub = n_pad // NS

    cp = pltpu.CompilerParams()
    if "needs_layout_passes" in pltpu.CompilerParams.__dataclass_fields__:
        cp = dataclasses.replace(cp, needs_layout_passes=False)

    @functools.partial(
        pl.kernel,
        compiler_params=cp,
        out_type=[
            jax.ShapeDtypeStruct((NC, n_pad, d), jnp.float32),
            jax.ShapeDtypeStruct((NC * n_pad,), jnp.float32),
        ],
        mesh=mesh,
        scratch_types=[
            pltpu.VMEM((CH,), jnp.int32),
            pltpu.VMEM((CH,), jnp.int32),
            pltpu.VMEM((CH, d), jnp.float32),
            pltpu.VMEM((n_pad,), jnp.float32),
            pltpu.VMEM((rows_per_sub,), jnp.float32),
            pltpu.VMEM((rows_per_sub,), jnp.float32),
            pltpu.VMEM_SHARED((n_pad, d), jnp.float32),
            pltpu.VMEM_SHARED((NS, n_pad), jnp.float32),
            pltpu.SemaphoreType.DMA,
        ],
    )
    def seg_sum(src_hbm, dst_hbm, x_hbm, zx_hbm, accx_hbm, cnt_hbm,
                sidx_v, didx_v, rows_v, cnt_tile, red_v, rin_v,
                accx_sh, cstage_sh, sem):
        c = lax.axis_index("c")
        s = lax.axis_index("s")
        rbase = s * rows_per_sub

        z16 = jnp.zeros((16,), jnp.float32)
        one16 = jnp.full((16,), 1.0, jnp.float32)

        @pl.loop(0, n_pad, step=16)
        def _(r):
            cnt_tile.at[pl.ds(r, 16)][...] = z16

        pltpu.sync_copy(zx_hbm, accx_sh.at[pl.ds(rbase, rows_per_sub)])
        plsc.subcore_barrier()

        wbase = (c * NS + s) * epw

        @pl.loop(0, epw, step=CH)
        def _(i):
            base = wbase + i
            pltpu.sync_copy(src_hbm.at[pl.ds(base, CH)], sidx_v)
            pltpu.sync_copy(dst_hbm.at[pl.ds(base, CH)], didx_v)
            pltpu.async_copy(x_hbm.at[sidx_v], rows_v, sem).wait()
            pltpu.sync_copy(rows_v, accx_sh.at[didx_v], add=True)
            @pl.loop(0, CH, step=16)
            def _(j):
                idx16 = didx_v.at[pl.ds(j, 16)][...]
                plsc.addupdate_scatter(cnt_tile, [idx16], one16)

        plsc.subcore_barrier()

        pltpu.sync_copy(accx_sh.at[pl.ds(rbase, rows_per_sub)],
                        accx_hbm.at[c, pl.ds(rbase, rows_per_sub)])

        pltpu.sync_copy(cnt_tile, cstage_sh.at[s])
        plsc.subcore_barrier()

        @pl.loop(0, rows_per_sub, step=16)
        def _(r):
            red_v.at[pl.ds(r, 16)][...] = z16

        for k in range(NS):
            pltpu.sync_copy(cstage_sh.at[k, pl.ds(rbase, rows_per_sub)], rin_v)

            @pl.loop(0, rows_per_sub, step=16)
            def _(r):
                red_v.at[pl.ds(r, 16)][...] = (
                    red_v.at[pl.ds(r, 16)][...] + rin_v.at[pl.ds(r, 16)][...])

        pltpu.sync_copy(red_v,
                        cnt_hbm.at[pl.ds(c * n_pad + rbase, rows_per_sub)])

    return seg_sum(src, dst, x2d, zx)


def _tc_finish(x2d, a0, a1, c0, c1, degf, wsT, wnT, bs, bn, n, d, rows):
    grid = (n // rows,)

    def body(x_ref, a0_ref, a1_ref, c0_ref, c1_ref, deg_ref,
             wsT_ref, wnT_ref, bs_ref, bn_ref, o_ref):
        xv = x_ref[...]
        acc = a0_ref[...] + a1_ref[...]
        cnt = c0_ref[...] + c1_ref[...]
        denom = jnp.maximum(deg_ref[...], 1.0)
        inv = 1.0 / denom
        self_out = jnp.dot(xv, wsT_ref[...],
                           preferred_element_type=jnp.float32,
                           precision=lax.Precision.HIGHEST) + bs_ref[...]
        neigh = jnp.dot(acc, wnT_ref[...],
                        preferred_element_type=jnp.float32,
                        precision=lax.Precision.HIGHEST)
        out = self_out + (neigh + cnt * bn_ref[...]) * inv
        o_ref[...] = jnp.where(out >= 0, out, 0.1 * out)

    return pl.pallas_call(
        body,
        grid=grid,
        in_specs=[
            pl.BlockSpec((rows, d), lambda i: (i, 0)),
            pl.BlockSpec((rows, d), lambda i: (i, 0)),
            pl.BlockSpec((rows, d), lambda i: (i, 0)),
            pl.BlockSpec((rows, 1), lambda i: (i, 0)),
            pl.BlockSpec((rows, 1), lambda i: (i, 0)),
            pl.BlockSpec((rows, 1), lambda i: (i, 0)),
            pl.BlockSpec((d, d), lambda i: (0, 0)),
            pl.BlockSpec((d, d), lambda i: (0, 0)),
            pl.BlockSpec((1, d), lambda i: (0, 0)),
            pl.BlockSpec((1, d), lambda i: (0, 0)),
        ],
        out_specs=pl.BlockSpec((rows, d), lambda i: (i, 0)),
        out_shape=jax.ShapeDtypeStruct((n, d), jnp.float32),
    )(x2d, a0, a1, c0, c1, degf, wsT, wnT, bs, bn)


def kernel(x, edge_index, deg, W_self, b_self, W_neigh, b_neigh):
    b, n, d = x.shape
    e = edge_index.shape[1]
    assert b == 1 and e % NW == 0 and (e // NW) % CH == 0

    src = edge_index[0].astype(jnp.int32)
    dst = edge_index[1].astype(jnp.int32)
    x2d = x.reshape(n, d).astype(jnp.float32)

    n_pad = ((n + NS * CH - 1) // (NS * CH)) * (NS * CH)
    rows_per_sub = n_pad // NS
    zx = jnp.zeros((rows_per_sub, d), jnp.float32)

    accx, cnt = _sc_segment_sum(src, dst, x2d, zx, n_pad)

    degf = deg.astype(jnp.float32).reshape(n, 1)
    out2d = _tc_finish(
        x2d, accx[0, :n], accx[1, :n],
        cnt[:n].reshape(n, 1), cnt[n_pad:n_pad + n].reshape(n, 1), degf,
        W_self.T.astype(jnp.float32), W_neigh.T.astype(jnp.float32),
        b_self.reshape(1, d).astype(jnp.float32),
        b_neigh.reshape(1, d).astype(jnp.float32),
        n, d, rows=1000,
    )
    return out2d.reshape(b, n, d).astype(x.dtype)


# speedup vs baseline: 8.1424x; 1.6454x over previous
import dataclasses
import functools

import jax
import jax.numpy as jnp
from jax import lax
from jax.experimental import pallas as pl
from jax.experimental.pallas import tpu as pltpu
from jax.experimental.pallas import tpu_sc as plsc

NC = 2
NS = 16
NW = NC * NS
CH = 128


def _sc_segment_sum(src, dst, x2d, zx, n_pad):
    e = src.shape[0]
    d = x2d.shape[1]
    epw = e // NW
    cpw = epw // CH
    mesh = plsc.VectorSubcoreMesh(core_axis_name="c", subcore_axis_name="s", num_cores=NC)
    rows_per_sub = n_pad // NS

    cp = pltpu.CompilerParams()
    if "needs_layout_passes" in pltpu.CompilerParams.__dataclass_fields__:
        cp = dataclasses.replace(cp, needs_layout_passes=False)

    @functools.partial(
        pl.kernel,
        compiler_params=cp,
        out_type=[
            jax.ShapeDtypeStruct((NC, n_pad, d), jnp.float32),
            jax.ShapeDtypeStruct((NC * n_pad,), jnp.float32),
            jax.ShapeDtypeStruct((NW * n_pad,), jnp.float32),
        ],
        mesh=mesh,
        scratch_types=[
            pltpu.VMEM((CH,), jnp.int32),
            pltpu.VMEM((CH,), jnp.int32),
            pltpu.VMEM((CH,), jnp.int32),
            pltpu.VMEM((CH,), jnp.int32),
            pltpu.VMEM((CH, d), jnp.float32),
            pltpu.VMEM((CH, d), jnp.float32),
            pltpu.VMEM((n_pad,), jnp.float32),
            pltpu.VMEM((rows_per_sub,), jnp.float32),
            pltpu.VMEM((rows_per_sub,), jnp.float32),
            pltpu.VMEM_SHARED((n_pad, d), jnp.float32),
            pltpu.SemaphoreType.DMA,
            pltpu.SemaphoreType.DMA,
        ],
    )
    def seg_sum(src_hbm, dst_hbm, x_hbm, zx_hbm, accx_hbm, cnt_hbm, cpart_hbm,
                sidx0_v, sidx1_v, didx0_v, didx1_v, rows0_v, rows1_v,
                cnt_tile, red_v, rin_v, accx_sh, sem0, sem1):
        c = lax.axis_index("c")
        s = lax.axis_index("s")
        rbase = s * rows_per_sub
        wbase = (c * NS + s) * epw

        pltpu.sync_copy(src_hbm.at[pl.ds(wbase, CH)], sidx0_v)
        pltpu.sync_copy(dst_hbm.at[pl.ds(wbase, CH)], didx0_v)
        pltpu.make_async_copy(x_hbm.at[sidx0_v], rows0_v, sem0).start()

        z16 = jnp.zeros((16,), jnp.float32)
        one16 = jnp.full((16,), 1.0, jnp.float32)

        @pl.loop(0, n_pad, step=16)
        def _(r):
            cnt_tile.at[pl.ds(r, 16)][...] = z16

        pltpu.sync_copy(zx_hbm, accx_sh.at[pl.ds(rbase, rows_per_sub)])
        plsc.subcore_barrier()

        def do_chunk(didx, buf):
            pltpu.sync_copy(buf, accx_sh.at[didx], add=True)

            @pl.loop(0, CH, step=16)
            def _(j):
                idx16 = didx.at[pl.ds(j, 16)][...]
                plsc.addupdate_scatter(cnt_tile, [idx16], one16)

        @pl.loop(0, cpw, step=2)
        def _(t):
            base = wbase + t * CH
            pltpu.sync_copy(src_hbm.at[pl.ds(base + CH, CH)], sidx1_v)
            pltpu.sync_copy(dst_hbm.at[pl.ds(base + CH, CH)], didx1_v)
            pltpu.make_async_copy(x_hbm.at[sidx1_v], rows1_v, sem1).start()
            pltpu.make_async_copy(x_hbm.at[sidx0_v], rows0_v, sem0).wait()
            do_chunk(didx0_v, rows0_v)

            @pl.when(t + 2 < cpw)
            def _():
                pltpu.sync_copy(src_hbm.at[pl.ds(base + 2 * CH, CH)], sidx0_v)
                pltpu.sync_copy(dst_hbm.at[pl.ds(base + 2 * CH, CH)], didx0_v)
                pltpu.make_async_copy(x_hbm.at[sidx0_v], rows0_v, sem0).start()

            pltpu.make_async_copy(x_hbm.at[sidx1_v], rows1_v, sem1).wait()
            do_chunk(didx1_v, rows1_v)

        plsc.subcore_barrier()

        pltpu.sync_copy(accx_sh.at[pl.ds(rbase, rows_per_sub)],
                        accx_hbm.at[c, pl.ds(rbase, rows_per_sub)])

        pltpu.sync_copy(cnt_tile,
                        cpart_hbm.at[pl.ds((c * NS + s) * n_pad, n_pad)])
        plsc.subcore_barrier()

        @pl.loop(0, rows_per_sub, step=16)
        def _(r):
            red_v.at[pl.ds(r, 16)][...] = z16

        for k in range(NS):
            pltpu.sync_copy(
                cpart_hbm.at[pl.ds((c * NS + k) * n_pad + rbase, rows_per_sub)],
                rin_v)

            @pl.loop(0, rows_per_sub, step=16)
            def _(r):
                red_v.at[pl.ds(r, 16)][...] = (
                    red_v.at[pl.ds(r, 16)][...] + rin_v.at[pl.ds(r, 16)][...])

        pltpu.sync_copy(red_v,
                        cnt_hbm.at[pl.ds(c * n_pad + rbase, rows_per_sub)])

    return seg_sum(src, dst, x2d, zx)


def _tc_finish(x2d, a0, a1, c0, c1, degf, wsT, wnT, bs, bn, n, d, rows):
    grid = (n // rows,)

    def body(x_ref, a0_ref, a1_ref, c0_ref, c1_ref, deg_ref,
             wsT_ref, wnT_ref, bs_ref, bn_ref, o_ref):
        xv = x_ref[...]
        acc = a0_ref[...] + a1_ref[...]
        cnt = c0_ref[...] + c1_ref[...]
        denom = jnp.maximum(deg_ref[...], 1.0)
        inv = 1.0 / denom
        self_out = jnp.dot(xv, wsT_ref[...],
                           preferred_element_type=jnp.float32,
                           precision=lax.Precision.HIGHEST) + bs_ref[...]
        neigh = jnp.dot(acc, wnT_ref[...],
                        preferred_element_type=jnp.float32,
                        precision=lax.Precision.HIGHEST)
        out = self_out + (neigh + cnt * bn_ref[...]) * inv
        o_ref[...] = jnp.where(out >= 0, out, 0.1 * out)

    return pl.pallas_call(
        body,
        grid=grid,
        in_specs=[
            pl.BlockSpec((rows, d), lambda i: (i, 0)),
            pl.BlockSpec((rows, d), lambda i: (i, 0)),
            pl.BlockSpec((rows, d), lambda i: (i, 0)),
            pl.BlockSpec((rows, 1), lambda i: (i, 0)),
            pl.BlockSpec((rows, 1), lambda i: (i, 0)),
            pl.BlockSpec((rows, 1), lambda i: (i, 0)),
            pl.BlockSpec((d, d), lambda i: (0, 0)),
            pl.BlockSpec((d, d), lambda i: (0, 0)),
            pl.BlockSpec((1, d), lambda i: (0, 0)),
            pl.BlockSpec((1, d), lambda i: (0, 0)),
        ],
        out_specs=pl.BlockSpec((rows, d), lambda i: (i, 0)),
        out_shape=jax.ShapeDtypeStruct((n, d), jnp.float32),
    )(x2d, a0, a1, c0, c1, degf, wsT, wnT, bs, bn)


def kernel(x, edge_index, deg, W_self, b_self, W_neigh, b_neigh):
    b, n, d = x.shape
    e = edge_index.shape[1]
    assert b == 1

    src = edge_index[0].astype(jnp.int32)
    dst = edge_index[1].astype(jnp.int32)
    x2d = x.reshape(n, d).astype(jnp.float32)

    n_pad = ((n + 2047) // 2048) * 2048
    assert n_pad >= n + 64
    rows_per_sub = n_pad // NS
    zx = jnp.zeros((rows_per_sub, d), jnp.float32)

    quantum = NW * CH * 2
    e_pad = ((e + quantum - 1) // quantum) * quantum
    if e_pad != e:
        pad_i = jnp.arange(e_pad - e, dtype=jnp.int32) % 64
        src = jnp.concatenate([src, pad_i])
        dst = jnp.concatenate([dst, n + pad_i])

    accx, cnt, _unused_partials = _sc_segment_sum(src, dst, x2d, zx, n_pad)

    degf = deg.astype(jnp.float32).reshape(n, 1)
    out2d = _tc_finish(
        x2d, accx[0, :n], accx[1, :n],
        cnt[:n].reshape(n, 1), cnt[n_pad:n_pad + n].reshape(n, 1), degf,
        W_self.T.astype(jnp.float32), W_neigh.T.astype(jnp.float32),
        b_self.reshape(1, d).astype(jnp.float32),
        b_neigh.reshape(1, d).astype(jnp.float32),
        n, d, rows=1000,
    )
    return out2d.reshape(b, n, d).astype(x.dtype)


# speedup vs baseline: 10.2623x; 1.2604x over previous
import dataclasses
import functools

import jax
import jax.numpy as jnp
from jax import lax
from jax.experimental import pallas as pl
from jax.experimental.pallas import tpu as pltpu
from jax.experimental.pallas import tpu_sc as plsc

NC = 2
NS = 16
NW = NC * NS
CH = 64


def _sc_segment_sum(src, dst, x2d, zx, n_pad):
    e = src.shape[0]
    d = x2d.shape[1]
    epw = e // NW
    cpw = epw // CH
    mesh = plsc.VectorSubcoreMesh(core_axis_name="c", subcore_axis_name="s", num_cores=NC)
    rows_per_sub = n_pad // NS

    cp = pltpu.CompilerParams()
    if "needs_layout_passes" in pltpu.CompilerParams.__dataclass_fields__:
        cp = dataclasses.replace(cp, needs_layout_passes=False)

    @functools.partial(
        pl.kernel,
        compiler_params=cp,
        out_type=[
            jax.ShapeDtypeStruct((NC, n_pad, d), jnp.float32),
            jax.ShapeDtypeStruct((NC * n_pad,), jnp.float32),
            jax.ShapeDtypeStruct((NW * n_pad,), jnp.float32),
        ],
        mesh=mesh,
        scratch_types=(
            [pltpu.VMEM((CH,), jnp.int32)] * 4
            + [pltpu.VMEM((CH,), jnp.int32)] * 4
            + [pltpu.VMEM((CH, d), jnp.float32)] * 4
            + [
                pltpu.VMEM((n_pad,), jnp.float32),
                pltpu.VMEM((rows_per_sub,), jnp.float32),
                pltpu.VMEM((rows_per_sub,), jnp.float32),
                pltpu.VMEM_SHARED((n_pad, d), jnp.float32),
            ]
            + [pltpu.SemaphoreType.DMA] * 16
        ),
    )
    def seg_sum(src_hbm, dst_hbm, x_hbm, zx_hbm, accx_hbm, cnt_hbm, cpart_hbm,
                *refs):
        sidx_b = refs[0:4]
        didx_b = refs[4:8]
        rows_b = refs[8:12]
        cnt_tile, red_v, rin_v, accx_sh = refs[12:16]
        gsem = refs[16:20]
        issem = refs[20:24]
        idsem = refs[24:28]
        ssem = refs[28:32]

        c = lax.axis_index("c")
        s = lax.axis_index("s")
        rbase = s * rows_per_sub
        wbase = (c * NS + s) * epw

        z16 = jnp.zeros((16,), jnp.float32)
        one16 = jnp.full((16,), 1.0, jnp.float32)

        def a_sidx(base, u):
            pltpu.make_async_copy(
                src_hbm.at[pl.ds(base, CH)], sidx_b[u], issem[u]).start()

        def a_didx(base, u):
            pltpu.make_async_copy(
                dst_hbm.at[pl.ds(base, CH)], didx_b[u], idsem[u]).start()

        def b_gather(base, u, wait_scatter):
            if wait_scatter:
                pltpu.make_async_copy(
                    rows_b[u], accx_sh.at[didx_b[u]], ssem[u]).wait()
            pltpu.make_async_copy(
                src_hbm.at[pl.ds(base, CH)], sidx_b[u], issem[u]).wait()
            pltpu.make_async_copy(
                x_hbm.at[sidx_b[u]], rows_b[u], gsem[u]).start()

        def c_consume(base, u):
            pltpu.make_async_copy(
                dst_hbm.at[pl.ds(base, CH)], didx_b[u], idsem[u]).wait()
            pltpu.make_async_copy(
                x_hbm.at[sidx_b[u]], rows_b[u], gsem[u]).wait()
            pltpu.make_async_copy(
                rows_b[u], accx_sh.at[didx_b[u]], ssem[u]).start(add=True)

            @pl.loop(0, CH, step=16)
            def _(j):
                idx16 = didx_b[u].at[pl.ds(j, 16)][...]
                plsc.addupdate_scatter(cnt_tile, [idx16], one16)

        for k in range(4):
            a_sidx(wbase + k * CH, k % 4)
        for k in range(3):
            a_didx(wbase + k * CH, k % 4)

        @pl.loop(0, n_pad, step=16)
        def _(r):
            cnt_tile.at[pl.ds(r, 16)][...] = z16

        pltpu.sync_copy(zx_hbm, accx_sh.at[pl.ds(rbase, rows_per_sub)])
        plsc.subcore_barrier()

        for k in range(3):
            b_gather(wbase + k * CH, k % 4, wait_scatter=False)

        def slot(k_dyn, u, wait_b):
            c_consume(k_dyn, u)
            a_sidx(k_dyn + 4 * CH, u)
            b_gather(k_dyn + 3 * CH, (u + 3) % 4, wait_scatter=wait_b)
            a_didx(k_dyn + 3 * CH, (u + 3) % 4)

        for u in range(4):
            slot(wbase + u * CH, u, wait_b=(u >= 1))

        @pl.loop(4, cpw - 4, step=4)
        def _(t):
            base = wbase + t * CH
            for u in range(4):
                slot(base + u * CH, u, wait_b=True)

        for u in range(4):
            k = cpw - 4 + u
            c_consume(wbase + k * CH, u)
            if k + 3 < cpw:
                b_gather(wbase + (k + 3) * CH, (u + 3) % 4, wait_scatter=True)
                a_didx(wbase + (k + 3) * CH, (u + 3) % 4)
        for u in range(4):
            pltpu.make_async_copy(
                rows_b[u], accx_sh.at[didx_b[u]], ssem[u]).wait()
        plsc.subcore_barrier()

        pltpu.sync_copy(accx_sh.at[pl.ds(rbase, rows_per_sub)],
                        accx_hbm.at[c, pl.ds(rbase, rows_per_sub)])

        pltpu.sync_copy(cnt_tile,
                        cpart_hbm.at[pl.ds((c * NS + s) * n_pad, n_pad)])
        plsc.subcore_barrier()

        @pl.loop(0, rows_per_sub, step=16)
        def _(r):
            red_v.at[pl.ds(r, 16)][...] = z16

        for k in range(NS):
            pltpu.sync_copy(
                cpart_hbm.at[pl.ds((c * NS + k) * n_pad + rbase, rows_per_sub)],
                rin_v)

            @pl.loop(0, rows_per_sub, step=16)
            def _(r):
                red_v.at[pl.ds(r, 16)][...] = (
                    red_v.at[pl.ds(r, 16)][...] + rin_v.at[pl.ds(r, 16)][...])

        pltpu.sync_copy(red_v,
                        cnt_hbm.at[pl.ds(c * n_pad + rbase, rows_per_sub)])

    return seg_sum(src, dst, x2d, zx)


def _tc_finish(x2d, a0, a1, c0, c1, degf, wsT, wnT, bs, bn, n, d, rows):
    grid = (n // rows,)

    def body(x_ref, a0_ref, a1_ref, c0_ref, c1_ref, deg_ref,
             wsT_ref, wnT_ref, bs_ref, bn_ref, o_ref):
        xv = x_ref[...]
        acc = a0_ref[...] + a1_ref[...]
        cnt = c0_ref[...] + c1_ref[...]
        denom = jnp.maximum(deg_ref[...], 1.0)
        inv = 1.0 / denom
        self_out = jnp.dot(xv, wsT_ref[...],
                           preferred_element_type=jnp.float32,
                           precision=lax.Precision.HIGHEST) + bs_ref[...]
        neigh = jnp.dot(acc, wnT_ref[...],
                        preferred_element_type=jnp.float32,
                        precision=lax.Precision.HIGHEST)
        out = self_out + (neigh + cnt * bn_ref[...]) * inv
        o_ref[...] = jnp.where(out >= 0, out, 0.1 * out)

    return pl.pallas_call(
        body,
        grid=grid,
        in_specs=[
            pl.BlockSpec((rows, d), lambda i: (i, 0)),
            pl.BlockSpec((rows, d), lambda i: (i, 0)),
            pl.BlockSpec((rows, d), lambda i: (i, 0)),
            pl.BlockSpec((rows, 1), lambda i: (i, 0)),
            pl.BlockSpec((rows, 1), lambda i: (i, 0)),
            pl.BlockSpec((rows, 1), lambda i: (i, 0)),
            pl.BlockSpec((d, d), lambda i: (0, 0)),
            pl.BlockSpec((d, d), lambda i: (0, 0)),
            pl.BlockSpec((1, d), lambda i: (0, 0)),
            pl.BlockSpec((1, d), lambda i: (0, 0)),
        ],
        out_specs=pl.BlockSpec((rows, d), lambda i: (i, 0)),
        out_shape=jax.ShapeDtypeStruct((n, d), jnp.float32),
    )(x2d, a0, a1, c0, c1, degf, wsT, wnT, bs, bn)


def kernel(x, edge_index, deg, W_self, b_self, W_neigh, b_neigh):
    b, n, d = x.shape
    e = edge_index.shape[1]
    assert b == 1

    src = edge_index[0].astype(jnp.int32)
    dst = edge_index[1].astype(jnp.int32)
    x2d = x.reshape(n, d).astype(jnp.float32)

    n_pad = ((n + 2047) // 2048) * 2048
    assert n_pad >= n + 64
    rows_per_sub = n_pad // NS
    zx = jnp.zeros((rows_per_sub, d), jnp.float32)

    quantum = NW * CH * 4
    e_pad = ((e + quantum - 1) // quantum) * quantum
    e_pad = max(e_pad, 2 * quantum)
    if e_pad != e:
        pad_i = jnp.arange(e_pad - e, dtype=jnp.int32) % 64
        src = jnp.concatenate([src, pad_i])
        dst = jnp.concatenate([dst, n + pad_i])

    accx, cnt, _unused_partials = _sc_segment_sum(src, dst, x2d, zx, n_pad)

    degf = deg.astype(jnp.float32).reshape(n, 1)
    out2d = _tc_finish(
        x2d, accx[0, :n], accx[1, :n],
        cnt[:n].reshape(n, 1), cnt[n_pad:n_pad + n].reshape(n, 1), degf,
        W_self.T.astype(jnp.float32), W_neigh.T.astype(jnp.float32),
        b_self.reshape(1, d).astype(jnp.float32),
        b_neigh.reshape(1, d).astype(jnp.float32),
        n, d, rows=1000,
    )
    return out2d.reshape(b, n, d).astype(x.dtype)


# speedup vs baseline: 10.8724x; 1.0594x over previous
import dataclasses
import functools

import jax
import jax.numpy as jnp
from jax import lax
from jax.experimental import pallas as pl
from jax.experimental.pallas import tpu as pltpu
from jax.experimental.pallas import tpu_sc as plsc

NC = 2
NS = 16
NW = NC * NS
CH = 64


def _sc_segment_sum(src, dst, x2d, zx, n_pad):
    e = src.shape[0]
    d = x2d.shape[1]
    epw = e // NW
    cpw = epw // CH
    mesh = plsc.VectorSubcoreMesh(core_axis_name="c", subcore_axis_name="s", num_cores=NC)
    rows_per_sub = n_pad // NS

    cp = pltpu.CompilerParams()
    if "needs_layout_passes" in pltpu.CompilerParams.__dataclass_fields__:
        cp = dataclasses.replace(cp, needs_layout_passes=False)

    @functools.partial(
        pl.kernel,
        compiler_params=cp,
        out_type=[
            jax.ShapeDtypeStruct((NC, n_pad, d), jnp.float32),
            jax.ShapeDtypeStruct((NC * n_pad,), jnp.float32),
            jax.ShapeDtypeStruct((NW * n_pad,), jnp.float32),
        ],
        mesh=mesh,
        scratch_types=(
            [pltpu.VMEM((CH,), jnp.int32)] * 4
            + [pltpu.VMEM((CH,), jnp.int32)] * 4
            + [pltpu.VMEM((CH, d), jnp.float32)] * 4
            + [
                pltpu.VMEM((n_pad,), jnp.float32),
                pltpu.VMEM((rows_per_sub,), jnp.float32),
                pltpu.VMEM((rows_per_sub,), jnp.float32),
                pltpu.VMEM_SHARED((n_pad, d), jnp.float32),
            ]
            + [pltpu.SemaphoreType.DMA] * 16
        ),
    )
    def seg_sum(src_hbm, dst_hbm, x_hbm, zx_hbm, accx_hbm, cnt_hbm, cpart_hbm,
                *refs):
        sidx_b = refs[0:4]
        didx_b = refs[4:8]
        rows_b = refs[8:12]
        cnt_tile, red_v, rin_v, accx_sh = refs[12:16]
        gsem = refs[16:20]
        issem = refs[20:24]
        idsem = refs[24:28]
        ssem = refs[28:32]

        c = lax.axis_index("c")
        s = lax.axis_index("s")
        rbase = s * rows_per_sub
        wbase = (c * NS + s) * epw

        z16 = jnp.zeros((16,), jnp.float32)
        one16 = jnp.full((16,), 1.0, jnp.float32)

        def a_sidx(base, u):
            pltpu.make_async_copy(
                src_hbm.at[pl.ds(base, CH)], sidx_b[u], issem[u]).start()

        def a_didx(base, u):
            pltpu.make_async_copy(
                dst_hbm.at[pl.ds(base, CH)], didx_b[u], idsem[u]).start()

        def b_gather(base, u, wait_scatter):
            if wait_scatter:
                pltpu.make_async_copy(
                    rows_b[u], accx_sh.at[didx_b[u]], ssem[u]).wait()
            pltpu.make_async_copy(
                src_hbm.at[pl.ds(base, CH)], sidx_b[u], issem[u]).wait()
            pltpu.make_async_copy(
                x_hbm.at[sidx_b[u]], rows_b[u], gsem[u]).start()

        def c_consume(base, u):
            pltpu.make_async_copy(
                dst_hbm.at[pl.ds(base, CH)], didx_b[u], idsem[u]).wait()
            pltpu.make_async_copy(
                x_hbm.at[sidx_b[u]], rows_b[u], gsem[u]).wait()
            pltpu.make_async_copy(
                rows_b[u], accx_sh.at[didx_b[u]], ssem[u]).start(add=True)

            @pl.loop(0, CH, step=16)
            def _(j):
                idx16 = didx_b[u].at[pl.ds(j, 16)][...]
                plsc.addupdate_scatter(cnt_tile, [idx16], one16)

        for k in range(4):
            a_sidx(wbase + k * CH, k % 4)
        for k in range(3):
            a_didx(wbase + k * CH, k % 4)

        @pl.loop(0, n_pad, step=16)
        def _(r):
            cnt_tile.at[pl.ds(r, 16)][...] = z16

        pltpu.sync_copy(zx_hbm, accx_sh.at[pl.ds(rbase, rows_per_sub)])
        plsc.subcore_barrier()

        for k in range(3):
            b_gather(wbase + k * CH, k % 4, wait_scatter=False)

        def slot(k_dyn, u, wait_b):
            c_consume(k_dyn, u)
            a_sidx(k_dyn + 4 * CH, u)
            b_gather(k_dyn + 3 * CH, (u + 3) % 4, wait_scatter=wait_b)
            a_didx(k_dyn + 3 * CH, (u + 3) % 4)

        for u in range(4):
            slot(wbase + u * CH, u, wait_b=(u >= 1))

        @pl.loop(4, cpw - 4, step=4)
        def _(t):
            base = wbase + t * CH
            for u in range(4):
                slot(base + u * CH, u, wait_b=True)

        for u in range(4):
            k = cpw - 4 + u
            c_consume(wbase + k * CH, u)
            if k + 3 < cpw:
                b_gather(wbase + (k + 3) * CH, (u + 3) % 4, wait_scatter=True)
                a_didx(wbase + (k + 3) * CH, (u + 3) % 4)
        for u in range(4):
            pltpu.make_async_copy(
                rows_b[u], accx_sh.at[didx_b[u]], ssem[u]).wait()
        plsc.subcore_barrier()

        pltpu.sync_copy(accx_sh.at[pl.ds(rbase, rows_per_sub)],
                        accx_hbm.at[c, pl.ds(rbase, rows_per_sub)])

        pltpu.sync_copy(cnt_tile,
                        cpart_hbm.at[pl.ds((c * NS + s) * n_pad, n_pad)])
        plsc.subcore_barrier()

        @pl.loop(0, rows_per_sub, step=16)
        def _(r):
            red_v.at[pl.ds(r, 16)][...] = z16

        for k in range(NS):
            pltpu.sync_copy(
                cpart_hbm.at[pl.ds((c * NS + k) * n_pad + rbase, rows_per_sub)],
                rin_v)

            @pl.loop(0, rows_per_sub, step=16)
            def _(r):
                red_v.at[pl.ds(r, 16)][...] = (
                    red_v.at[pl.ds(r, 16)][...] + rin_v.at[pl.ds(r, 16)][...])

        pltpu.sync_copy(red_v,
                        cnt_hbm.at[pl.ds(c * n_pad + rbase, rows_per_sub)])

    return seg_sum(src, dst, x2d, zx)


def _tc_self(x2d, wsT, bs, n, d, rows):
    def body(x_ref, wsT_ref, bs_ref, o_ref):
        o_ref[...] = jnp.dot(x_ref[...], wsT_ref[...],
                             preferred_element_type=jnp.float32) + bs_ref[...]

    return pl.pallas_call(
        body,
        grid=(n // rows,),
        in_specs=[
            pl.BlockSpec((rows, d), lambda i: (i, 0)),
            pl.BlockSpec((d, d), lambda i: (0, 0)),
            pl.BlockSpec((1, d), lambda i: (0, 0)),
        ],
        out_specs=pl.BlockSpec((rows, d), lambda i: (i, 0)),
        out_shape=jax.ShapeDtypeStruct((n, d), jnp.float32),
    )(x2d, wsT, bs)


def _tc_finish(self_out, accx, cnt3, degf, wnT, bn, n, n_pad, d, rows):
    grid = (n // rows,)

    def body(s_ref, a_ref, c_ref, deg_ref, wnT_ref, bn_ref, o_ref):
        acc = a_ref[0] + a_ref[1]
        cnt = c_ref[0] + c_ref[1]
        denom = jnp.maximum(deg_ref[...], 1.0)
        inv = 1.0 / denom
        neigh = jnp.dot(acc, wnT_ref[...], preferred_element_type=jnp.float32)
        out = s_ref[...] + (neigh + cnt * bn_ref[...]) * inv
        o_ref[...] = jnp.where(out >= 0, out, 0.1 * out)

    return pl.pallas_call(
        body,
        grid=grid,
        in_specs=[
            pl.BlockSpec((rows, d), lambda i: (i, 0)),
            pl.BlockSpec((NC, rows, d), lambda i: (0, i, 0)),
            pl.BlockSpec((NC, rows, 1), lambda i: (0, i, 0)),
            pl.BlockSpec((rows, 1), lambda i: (i, 0)),
            pl.BlockSpec((d, d), lambda i: (0, 0)),
            pl.BlockSpec((1, d), lambda i: (0, 0)),
        ],
        out_specs=pl.BlockSpec((rows, d), lambda i: (i, 0)),
        out_shape=jax.ShapeDtypeStruct((n, d), jnp.float32),
    )(self_out, accx, cnt3, degf, wnT, bn)


def kernel(x, edge_index, deg, W_self, b_self, W_neigh, b_neigh):
    b, n, d = x.shape
    e = edge_index.shape[1]
    assert b == 1

    src = edge_index[0].astype(jnp.int32)
    dst = edge_index[1].astype(jnp.int32)
    x2d = x.reshape(n, d).astype(jnp.float32)

    n_pad = ((n + 2047) // 2048) * 2048
    assert n_pad >= n + 64
    rows_per_sub = n_pad // NS
    zx = jnp.zeros((rows_per_sub, d), jnp.float32)

    quantum = NW * CH * 4
    e_pad = ((e + quantum - 1) // quantum) * quantum
    e_pad = max(e_pad, 2 * quantum)
    if e_pad != e:
        pad_i = jnp.arange(e_pad - e, dtype=jnp.int32) % 64
        src = jnp.concatenate([src, pad_i])
        dst = jnp.concatenate([dst, n + pad_i])

    accx, cnt, _unused_partials = _sc_segment_sum(src, dst, x2d, zx, n_pad)

    self_out = _tc_self(x2d, W_self.T.astype(jnp.float32),
                        b_self.reshape(1, d).astype(jnp.float32), n, d, 2000)

    degf = deg.astype(jnp.float32).reshape(n, 1)
    out2d = _tc_finish(
        self_out, accx, cnt.reshape(NC, n_pad, 1), degf,
        W_neigh.T.astype(jnp.float32),
        b_neigh.reshape(1, d).astype(jnp.float32),
        n, n_pad, d, rows=1000,
    )
    return out2d.reshape(b, n, d).astype(x.dtype)
